# baseline (speedup 1.0000x reference)
import functools

import jax
import jax.numpy as jnp
from jax import lax
from jax.experimental import pallas as pl
from jax.experimental.pallas import tpu as pltpu
from jax.experimental.pallas import tpu_sc as plsc

MF_DIM = 32
BATCH = 16384
NC = 2
NS = 16
NW = NC * NS
B_PER_W = BATCH // NW
CHUNK = 128
NCHUNK = B_PER_W // CHUNK
GROUPS = B_PER_W // 16


def _gmf_body(ui_hbm, ii_hbm, ut_hbm, it_hbm, wb_hbm, out_hbm,
              idx_u, idx_i, rows_u, rows_i, wb_v, out_v, sem):
    c = lax.axis_index("c")
    s = lax.axis_index("s")
    wid = s * NC + c

    for j in range(NCHUNK):
        pltpu.sync_copy(ui_hbm.at[wid * NCHUNK + j], idx_u.at[j])
        pltpu.sync_copy(ii_hbm.at[wid * NCHUNK + j], idx_i.at[j])
    pltpu.sync_copy(wb_hbm, wb_v)

    copies = []
    for j in range(NCHUNK):
        copies.append(pltpu.async_copy(
            ut_hbm.at[idx_u.at[j]], rows_u.at[pl.ds(j * CHUNK, CHUNK)], sem))
        copies.append(pltpu.async_copy(
            it_hbm.at[idx_i.at[j]], rows_i.at[pl.ds(j * CHUNK, CHUNK)], sem))
    for cp in copies:
        cp.wait()

    ws = [plsc.load_gather(wb_v, [jnp.full((16,), d, jnp.int32)])
          for d in range(MF_DIM)]
    bv = plsc.load_gather(wb_v, [jnp.full((16,), MF_DIM, jnp.int32)])
    lanes = lax.iota(jnp.int32, 16)

    def g_body(g, carry):
        rows = g * 16 + lanes
        acc = bv
        for d in range(MF_DIM):
            dcol = jnp.full((16,), d, jnp.int32)
            u_d = plsc.load_gather(rows_u, [rows, dcol])
            v_d = plsc.load_gather(rows_i, [rows, dcol])
            acc = acc + u_d * v_d * ws[d]
        out_v[pl.ds(g * 16, 16)] = 1.0 / (1.0 + jnp.exp(-acc))
        return carry

    lax.fori_loop(0, GROUPS, g_body, 0)
    pltpu.sync_copy(out_v, out_hbm.at[pl.ds(wid * B_PER_W, B_PER_W)])


@functools.partial(
    pl.kernel,
    mesh=plsc.VectorSubcoreMesh(core_axis_name="c", subcore_axis_name="s"),
    out_type=jax.ShapeDtypeStruct((BATCH,), jnp.float32),
    compiler_params=pltpu.CompilerParams(
        needs_layout_passes=False, use_tc_tiling_on_sc=False),
    scratch_types=[
        pltpu.VMEM((NCHUNK, CHUNK), jnp.int32),
        pltpu.VMEM((NCHUNK, CHUNK), jnp.int32),
        pltpu.VMEM((B_PER_W, MF_DIM), jnp.float32),
        pltpu.VMEM((B_PER_W, MF_DIM), jnp.float32),
        pltpu.VMEM((48,), jnp.float32),
        pltpu.VMEM((B_PER_W,), jnp.float32),
        pltpu.SemaphoreType.DMA,
    ],
)
def _gmf_sc(*args):
    _gmf_body(*args)


def kernel(user_input, item_input, user_table, item_table, W, b):
    ui = user_input.astype(jnp.int32).reshape(NW * NCHUNK, CHUNK)
    ii = item_input.astype(jnp.int32).reshape(NW * NCHUNK, CHUNK)
    wb = jnp.concatenate([
        W.reshape(MF_DIM).astype(jnp.float32),
        b.reshape(1).astype(jnp.float32),
        jnp.zeros((15,), jnp.float32),
    ])
    out = _gmf_sc(ui, ii, user_table, item_table, wb)
    return out.reshape(BATCH, 1)
